# baseline (speedup 1.0000x reference)
import jax
import jax.numpy as jnp
from jax.experimental import pallas as pl
from jax.experimental.pallas import tpu as pltpu

_K = 64
_R = 8
_N = 32768
_W = 512
_NCH = _N // _W
_MIN_I32 = -2147483648


def _orderable(x):
    b = pltpu.bitcast(x, jnp.int32)
    return b ^ (jax.lax.shift_right_arithmetic(b, 31) & 0x7FFFFFFF)


def _topk_kernel(x_ref, res_ref, idx_ref, pos_ref):
    x3 = x_ref[...]
    key3 = _orderable(x3)

    def count_ge(c):
        s = jnp.where(key3 >= c, 1, 0)
        return jnp.sum(jnp.sum(s, axis=1), axis=1)[:, None, None]

    zero = jnp.zeros((_R, 1, 1), jnp.int32)
    prefix = jnp.where(count_ge(zero) >= _K, zero,
                       jnp.full((_R, 1, 1), _MIN_I32, jnp.int32))

    def bit_body(i, prefix):
        cand = prefix | jax.lax.shift_left(1, 30 - i)
        return jnp.where(count_ge(cand) >= _K, cand, prefix)

    t3 = jax.lax.fori_loop(0, 31, bit_body, prefix)

    gt3 = key3 > t3
    eq3 = key3 == t3
    n_gt = jnp.sum(jnp.sum(jnp.where(gt3, 1, 0), axis=1), axis=1)
    budget3 = (_K - n_gt).astype(jnp.float32)[:, None, None]

    gt_i = jnp.where(gt3, 1, 0)
    eq_i = jnp.where(eq3, 1, 0)
    cnt_gt = jnp.sum(gt_i, axis=2)
    cnt_eq = jnp.sum(eq_i, axis=2)

    def excl_prefix(v):
        lane = jax.lax.broadcasted_iota(jnp.int32, (_R, _NCH), 1)
        acc = v
        for s in (1, 2, 4, 8, 16, 32):
            acc = acc + jnp.where(lane >= s, pltpu.roll(acc, s, axis=1), 0)
        return acc - v

    base_gt = excl_prefix(cnt_gt).astype(jnp.float32)[:, :, None]
    base_eq = excl_prefix(cnt_eq).astype(jnp.float32)[:, :, None]

    one_bf = jnp.bfloat16(1)
    zero_bf = jnp.bfloat16(0)
    gt_bf = gt_i.astype(jnp.bfloat16)
    eq_bf = eq_i.astype(jnp.bfloat16)
    tri = (jax.lax.broadcasted_iota(jnp.int32, (_W, _W), 0)
           < jax.lax.broadcasted_iota(jnp.int32, (_W, _W), 1)).astype(jnp.bfloat16)
    a2 = jnp.concatenate([gt_bf, eq_bf], axis=0)
    p2 = jax.lax.dot_general(a2, tri, (((2,), (0,)), ((), ())),
                             preferred_element_type=jnp.float32)
    g_gt = p2[:_R] + base_gt
    g_eq = p2[_R:] + base_eq
    sel3 = gt3 | (eq3 & (g_eq < budget3))
    res_ref[...] = jnp.where(sel3, jnp.maximum(x3, 0.0), 0.0)
    pos3 = g_gt + jnp.minimum(g_eq, budget3)
    pos_ref[...] = jnp.where(sel3, pos3, -1.0)

    lane_w = jax.lax.broadcasted_iota(jnp.int32, (_R, _W), 1)
    p_iota = (jax.lax.broadcasted_iota(jnp.int32, (_R, _K, _W), 1)
              .astype(jnp.bfloat16))

    def chunk_body(c, acc):
        kc = _orderable(x_ref[:, pl.ds(c, 1), :].reshape(_R, _W))
        pc = pos_ref[:, pl.ds(c, 1), :].reshape(_R, _W).astype(jnp.bfloat16)
        oh = jnp.where(pc[:, None, :] == p_iota, one_bf, zero_bf)
        gidx = c * _W + lane_w
        planes = jnp.stack(
            [(kc & 255).astype(jnp.bfloat16),
             (jax.lax.shift_right_logical(kc, 8) & 255).astype(jnp.bfloat16),
             (jax.lax.shift_right_logical(kc, 16) & 255).astype(jnp.bfloat16),
             (jax.lax.shift_right_logical(kc, 24) & 255).astype(jnp.bfloat16),
             jax.lax.shift_right_logical(gidx, 8).astype(jnp.bfloat16),
             (gidx & 255).astype(jnp.bfloat16)],
            axis=1)
        got = jax.lax.dot_general(planes, oh, (((2,), (2,)), ((0,), (0,))),
                                  preferred_element_type=jnp.float32)
        return acc + got

    acc0 = jnp.zeros((_R, 6, _K), jnp.float32)
    acc = jax.lax.fori_loop(0, _NCH, chunk_body, acc0)

    accs = acc.astype(jnp.int32)
    ck = ((jax.lax.shift_left(accs[:, 3, :], 24))
          | (jax.lax.shift_left(accs[:, 2, :], 16))
          | (jax.lax.shift_left(accs[:, 1, :], 8))
          | accs[:, 0, :])
    m_i = jax.lax.broadcasted_iota(jnp.int32, (_R, _K, _K), 2)
    j_i = jax.lax.broadcasted_iota(jnp.int32, (_R, _K, _K), 1)
    km = ck[:, None, :]
    kj = ck[:, :, None]
    a_gt = jnp.where(km > kj, 1, 0)
    a_tie = jnp.where(km == kj, 1, 0) * jnp.where(m_i < j_i, 1, 0)
    ahead_bf = (a_gt + a_tie).astype(jnp.bfloat16)
    ones_m = jnp.zeros((_R, 1, _K), jnp.bfloat16) + one_bf
    rank = jax.lax.dot_general(ahead_bf, ones_m, (((2,), (2,)), ((0,), (0,))),
                               preferred_element_type=jnp.float32)
    rank_bf = rank.astype(jnp.bfloat16)[:, :, 0]
    p_bf = (jax.lax.broadcasted_iota(jnp.int32, (_R, _K, _K), 1)
            .astype(jnp.bfloat16))
    oh3 = jnp.where(rank_bf[:, None, :] == p_bf, one_bf, zero_bf)
    ipl = jnp.stack([accs[:, 4, :].astype(jnp.bfloat16),
                     accs[:, 5, :].astype(jnp.bfloat16)], axis=1)
    got3 = jax.lax.dot_general(ipl, oh3, (((2,), (2,)), ((0,), (0,))),
                               preferred_element_type=jnp.float32)
    goti = got3.astype(jnp.int32)
    idx_ref[...] = jax.lax.shift_left(goti[:, 0, :], 8) | goti[:, 1, :]


def kernel(x):
    rows, n = x.shape
    x3 = x.reshape(rows, _NCH, _W)
    result, idx = pl.pallas_call(
        _topk_kernel,
        grid=(rows // _R,),
        in_specs=[pl.BlockSpec((_R, _NCH, _W), lambda i: (i, 0, 0))],
        out_specs=[
            pl.BlockSpec((_R, _NCH, _W), lambda i: (i, 0, 0)),
            pl.BlockSpec((_R, _K), lambda i: (i, 0)),
        ],
        out_shape=[
            jax.ShapeDtypeStruct((rows, _NCH, _W), x.dtype),
            jax.ShapeDtypeStruct((rows, _K), jnp.int32),
        ],
        scratch_shapes=[pltpu.VMEM((_R, _NCH, _W), jnp.float32)],
    )(x3)
    return (result.reshape(rows, n), idx)


# speedup vs baseline: 1.0799x; 1.0799x over previous
import jax
import jax.numpy as jnp
from jax.experimental import pallas as pl
from jax.experimental.pallas import tpu as pltpu

_K = 64
_R = 8
_N = 32768
_W = 1024
_NCH = _N // _W
_MIN_I32 = -2147483648


def _orderable(x):
    b = pltpu.bitcast(x, jnp.int32)
    return b ^ (jax.lax.shift_right_arithmetic(b, 31) & 0x7FFFFFFF)


def _topk_kernel(x_ref, res_ref, idx_ref, pos_ref):
    x3 = x_ref[...]
    key3 = _orderable(x3)

    def count_ge(c):
        s = jnp.where(key3 >= c, 1, 0)
        return jnp.sum(jnp.sum(s, axis=1), axis=1)[:, None, None]

    zero = jnp.zeros((_R, 1, 1), jnp.int32)
    prefix = jnp.where(count_ge(zero) >= _K, zero,
                       jnp.full((_R, 1, 1), _MIN_I32, jnp.int32))

    def bit_body(i, prefix):
        cand = prefix | jax.lax.shift_left(1, 30 - i)
        return jnp.where(count_ge(cand) >= _K, cand, prefix)

    t3 = jax.lax.fori_loop(0, 31, bit_body, prefix)

    gt3 = key3 > t3
    eq3 = key3 == t3
    n_gt = jnp.sum(jnp.sum(jnp.where(gt3, 1, 0), axis=1), axis=1)
    budget3 = (_K - n_gt).astype(jnp.float32)[:, None, None]

    gt_i = jnp.where(gt3, 1, 0)
    eq_i = jnp.where(eq3, 1, 0)
    cnt_gt = jnp.sum(gt_i, axis=2)
    cnt_eq = jnp.sum(eq_i, axis=2)

    def excl_prefix(v):
        lane = jax.lax.broadcasted_iota(jnp.int32, (_R, _NCH), 1)
        acc = v
        for s in (1, 2, 4, 8, 16, 32):
            acc = acc + jnp.where(lane >= s, pltpu.roll(acc, s, axis=1), 0)
        return acc - v

    base_gt = excl_prefix(cnt_gt).astype(jnp.float32)[:, :, None]
    base_eq = excl_prefix(cnt_eq).astype(jnp.float32)[:, :, None]

    one_bf = jnp.bfloat16(1)
    zero_bf = jnp.bfloat16(0)
    gt_bf = gt_i.astype(jnp.bfloat16)
    eq_bf = eq_i.astype(jnp.bfloat16)
    tri = (jax.lax.broadcasted_iota(jnp.int32, (_W, _W), 0)
           < jax.lax.broadcasted_iota(jnp.int32, (_W, _W), 1)).astype(jnp.bfloat16)
    a2 = jnp.concatenate([gt_bf, eq_bf], axis=0)
    p2 = jax.lax.dot_general(a2, tri, (((2,), (0,)), ((), ())),
                             preferred_element_type=jnp.float32)
    g_gt = p2[:_R] + base_gt
    g_eq = p2[_R:] + base_eq
    sel3 = gt3 | (eq3 & (g_eq < budget3))
    res_ref[...] = jnp.where(sel3, jnp.maximum(x3, 0.0), 0.0)
    pos3 = g_gt + jnp.minimum(g_eq, budget3)
    pos_ref[...] = jnp.where(sel3, pos3, -1.0)

    lane_w = jax.lax.broadcasted_iota(jnp.int32, (_R, _W), 1)
    p_iota = (jax.lax.broadcasted_iota(jnp.int32, (_R, _K, _W), 1)
              .astype(jnp.bfloat16))

    def chunk_body(c, acc):
        kc = _orderable(x_ref[:, pl.ds(c, 1), :].reshape(_R, _W))
        pc = pos_ref[:, pl.ds(c, 1), :].reshape(_R, _W).astype(jnp.bfloat16)
        oh = jnp.where(pc[:, None, :] == p_iota, one_bf, zero_bf)
        gidx = c * _W + lane_w
        planes = jnp.stack(
            [(kc & 255).astype(jnp.bfloat16),
             (jax.lax.shift_right_logical(kc, 8) & 255).astype(jnp.bfloat16),
             (jax.lax.shift_right_logical(kc, 16) & 255).astype(jnp.bfloat16),
             (jax.lax.shift_right_logical(kc, 24) & 255).astype(jnp.bfloat16),
             jax.lax.shift_right_logical(gidx, 8).astype(jnp.bfloat16),
             (gidx & 255).astype(jnp.bfloat16)],
            axis=1)
        got = jax.lax.dot_general(planes, oh, (((2,), (2,)), ((0,), (0,))),
                                  preferred_element_type=jnp.float32)
        return acc + got

    acc0 = jnp.zeros((_R, 6, _K), jnp.float32)
    acc = jax.lax.fori_loop(0, _NCH, chunk_body, acc0)

    accs = acc.astype(jnp.int32)
    ck = ((jax.lax.shift_left(accs[:, 3, :], 24))
          | (jax.lax.shift_left(accs[:, 2, :], 16))
          | (jax.lax.shift_left(accs[:, 1, :], 8))
          | accs[:, 0, :])
    m_i = jax.lax.broadcasted_iota(jnp.int32, (_R, _K, _K), 2)
    j_i = jax.lax.broadcasted_iota(jnp.int32, (_R, _K, _K), 1)
    km = ck[:, None, :]
    kj = ck[:, :, None]
    a_gt = jnp.where(km > kj, 1, 0)
    a_tie = jnp.where(km == kj, 1, 0) * jnp.where(m_i < j_i, 1, 0)
    ahead_bf = (a_gt + a_tie).astype(jnp.bfloat16)
    ones_m = jnp.zeros((_R, 1, _K), jnp.bfloat16) + one_bf
    rank = jax.lax.dot_general(ahead_bf, ones_m, (((2,), (2,)), ((0,), (0,))),
                               preferred_element_type=jnp.float32)
    rank_bf = rank.astype(jnp.bfloat16)[:, :, 0]
    p_bf = (jax.lax.broadcasted_iota(jnp.int32, (_R, _K, _K), 1)
            .astype(jnp.bfloat16))
    oh3 = jnp.where(rank_bf[:, None, :] == p_bf, one_bf, zero_bf)
    ipl = jnp.stack([accs[:, 4, :].astype(jnp.bfloat16),
                     accs[:, 5, :].astype(jnp.bfloat16)], axis=1)
    got3 = jax.lax.dot_general(ipl, oh3, (((2,), (2,)), ((0,), (0,))),
                               preferred_element_type=jnp.float32)
    goti = got3.astype(jnp.int32)
    idx_ref[...] = jax.lax.shift_left(goti[:, 0, :], 8) | goti[:, 1, :]


def kernel(x):
    rows, n = x.shape
    x3 = x.reshape(rows, _NCH, _W)
    result, idx = pl.pallas_call(
        _topk_kernel,
        grid=(rows // _R,),
        in_specs=[pl.BlockSpec((_R, _NCH, _W), lambda i: (i, 0, 0))],
        out_specs=[
            pl.BlockSpec((_R, _NCH, _W), lambda i: (i, 0, 0)),
            pl.BlockSpec((_R, _K), lambda i: (i, 0)),
        ],
        out_shape=[
            jax.ShapeDtypeStruct((rows, _NCH, _W), x.dtype),
            jax.ShapeDtypeStruct((rows, _K), jnp.int32),
        ],
        scratch_shapes=[pltpu.VMEM((_R, _NCH, _W), jnp.float32)],
    )(x3)
    return (result.reshape(rows, n), idx)
